# baseline (speedup 1.0000x reference)
import functools

import jax
import jax.numpy as jnp
from jax import lax
from jax.experimental import pallas as pl
from jax.experimental.pallas import tpu as pltpu
from jax.experimental.pallas import tpu_sc as plsc

N_ATOMS = 1048576
N_EMB = 128

NC = 2
NS = 16
NW = NC * NS

G = N_ATOMS // 128
GPW = G // NW

OUTER = 8
U1 = 28
U2G = 4
NBUF1 = 4
NBUF2 = 2
H2 = 2 * U2G
P1_TOT = OUTER * U1
Q2_TOT = OUTER * H2
R2_AT = (3, 6, 9, 12, 15, 18, 21, 24)


def _sc_gather(table, x2d):
    mesh = plsc.VectorSubcoreMesh(core_axis_name="c", subcore_axis_name="s")

    @functools.partial(
        pl.kernel,
        mesh=mesh,
        out_type=jax.ShapeDtypeStruct((G, 128, N_EMB), jnp.float32),
        scratch_types=[
            pltpu.VMEM((GPW, 128), jnp.int32),
            pltpu.VMEM((NBUF1, 128, N_EMB), jnp.float32),
            pltpu.VMEM((NBUF2, 64, N_EMB), jnp.float32),
            pltpu.VMEM_SHARED((100, N_EMB), jnp.float32),
            pltpu.SemaphoreType.DMA,
            pltpu.SemaphoreType.DMA,
            pltpu.SemaphoreType.DMA,
            pltpu.SemaphoreType.DMA,
        ],
    )
    def body(table_hbm, idx_hbm, out_hbm, idx_v, rows1, rows2, table_sh,
             sem_g1, sem_s1, sem_g2, sem_s2):
        wid = lax.axis_index("s") * NC + lax.axis_index("c")
        base = wid * GPW

        @pl.when(lax.axis_index("s") == 0)
        def _():
            pltpu.sync_copy(table_hbm, table_sh)

        pltpu.sync_copy(idx_hbm.at[pl.ds(base, GPW)], idx_v)
        plsc.subcore_barrier()

        def g1(l, b):
            pltpu.async_copy(table_sh.at[idx_v.at[l]], rows1.at[b], sem_g1)

        def g1_wait(l, b):
            pltpu.make_async_copy(
                table_sh.at[idx_v.at[l]], rows1.at[b], sem_g1).wait()

        def s1(l, b):
            pltpu.async_copy(rows1.at[b], out_hbm.at[base + l], sem_s1)

        def s1_wait(l, b):
            pltpu.make_async_copy(
                rows1.at[b], out_hbm.at[base + l], sem_s1).wait()

        def g2(l, half, b):
            pltpu.async_copy(
                table_hbm.at[idx_v.at[l, pl.ds(half * 64, 64)]],
                rows2.at[b], sem_g2)

        def g2_wait(l, half, b):
            pltpu.make_async_copy(
                table_hbm.at[idx_v.at[l, pl.ds(half * 64, 64)]],
                rows2.at[b], sem_g2).wait()

        def s2(l, half, b):
            pltpu.async_copy(
                rows2.at[b], out_hbm.at[base + l, pl.ds(half * 64, 64)],
                sem_s2)

        def s2_wait(l, half, b):
            pltpu.make_async_copy(
                rows2.at[b], out_hbm.at[base + l, pl.ds(half * 64, 64)],
                sem_s2).wait()

        def r2_group(i, h):
            return i * 32 + U1 + (h // 2), h % 2

        g1(0, 0)
        g1(1, 1)
        l0, hf0 = r2_group(0, 0)
        g2(l0, hf0, 0)

        def ring2_slot(i, h):
            q = i * H2 + h
            l, hf = r2_group(i, h)
            g2_wait(l, hf, h % NBUF2)
            s2(l, hf, h % NBUF2)

            @pl.when(q >= 1)
            def _():
                if h >= 1:
                    lp, hp = r2_group(i, h - 1)
                else:
                    lp, hp = r2_group(i - 1, H2 - 1)
                s2_wait(lp, hp, (h + 1) % NBUF2)

            @pl.when(q + 1 < Q2_TOT)
            def _():
                if h + 1 < H2:
                    ln, hn = r2_group(i, h + 1)
                else:
                    ln, hn = r2_group(i + 1, 0)
                g2(ln, hn, (h + 1) % NBUF2)

        def step(i, carry):
            for u in range(U1):
                p = i * U1 + u
                l = i * 32 + u
                g1_wait(l, u % NBUF1)
                s1(l, u % NBUF1)

                @pl.when(p >= 2)
                def _():
                    lp = l - 2 if u >= 2 else l - 6
                    s1_wait(lp, (u + 2) % NBUF1)

                @pl.when(p + 2 < P1_TOT)
                def _():
                    ln = l + 2 if u + 2 < U1 else l + 6
                    g1(ln, (u + 2) % NBUF1)

                if u in R2_AT:
                    ring2_slot(i, R2_AT.index(u))
            return carry

        lax.fori_loop(0, OUTER, step, 0)

        i_last = OUTER - 1
        s1_wait(i_last * 32 + U1 - 2, (U1 - 2) % NBUF1)
        s1_wait(i_last * 32 + U1 - 1, (U1 - 1) % NBUF1)
        ld, hd = r2_group(i_last, H2 - 1)
        s2_wait(ld, hd, (H2 - 1) % NBUF2)

    return body(table, x2d)


def kernel(x, embedding_list):
    out = _sc_gather(embedding_list, x.reshape(G, 128))
    return out.reshape(N_ATOMS, N_EMB)
